# baseline (speedup 1.0000x reference)
import functools

import jax
import jax.numpy as jnp
from jax import lax
from jax.experimental import pallas as pl
from jax.experimental.pallas import tpu as pltpu
from jax.experimental.pallas import tpu_sc as plsc

N = 16384
K = 8192
D = 64
TB = 256

_NC, _NS = 2, 16
_NW = _NC * _NS
_BPW = N // _NW
_CHUNK = 128


def _main_body(x_ref, x2_ref, w2_ref, wt_ref, ids_ref,
               idx_ref, enc_ref, cnt_ref, acc_ref):
    i = pl.program_id(0)
    x = x_ref[...]
    mm = jnp.dot(x, wt_ref[...], preferred_element_type=jnp.float32)
    d = (x2_ref[...] + w2_ref[...]) - 2.0 * mm
    m = jnp.min(d, axis=1, keepdims=True)
    ids = ids_ref[...]
    idf = jnp.min(jnp.where(d == m, ids, jnp.float32(K)),
                  axis=1, keepdims=True)
    oh = jnp.where(ids == idf, jnp.float32(1.0), jnp.float32(0.0))
    enc_ref[...] = oh
    idx_ref[...] = idf.astype(jnp.int32)
    part = jnp.sum(oh, axis=0, keepdims=True)

    @pl.when(i == 0)
    def _():
        acc_ref[...] = part

    @pl.when(i > 0)
    def _():
        acc_ref[...] = acc_ref[...] + part

    @pl.when(i == N // TB - 1)
    def _():
        cnt_ref[...] = acc_ref[...]


FB = 2048


def _fin_body(x_ref, q_ref, cnt_ref, qst_ref, loss_ref, perp_ref, acc_ref):
    i = pl.program_id(0)
    x = x_ref[...]
    q = q_ref[...]
    qst_ref[...] = x + (q - x)
    dlt = q - x
    e = jnp.sum(jnp.sum(dlt * dlt, axis=1, keepdims=True), axis=0,
                keepdims=True)

    @pl.when(i == 0)
    def _():
        acc_ref[...] = e

    @pl.when(i > 0)
    def _():
        acc_ref[...] = acc_ref[...] + e

    @pl.when(i == N // FB - 1)
    def _():
        et = acc_ref[...] * (1.0 / (N * D))
        loss_ref[...] = et + 0.25 * et
        p = cnt_ref[...] * (1.0 / N)
        eps = jnp.float32(jnp.finfo(jnp.float32).eps)
        ent = jnp.sum(p * jnp.log(p + eps), axis=1, keepdims=True)
        perp_ref[...] = jnp.exp(-ent)


@functools.cache
def _make_sc_gather():
    mesh = plsc.VectorSubcoreMesh(
        core_axis_name="c", subcore_axis_name="s",
        num_cores=_NC, num_subcores=_NS)

    @functools.partial(
        pl.kernel,
        mesh=mesh,
        out_type=jax.ShapeDtypeStruct((N, D), jnp.float32),
        scratch_types=[
            pltpu.VMEM((_CHUNK,), jnp.int32),
            pltpu.VMEM((_CHUNK, D), jnp.float32),
            pltpu.SemaphoreType.DMA,
        ],
        compiler_params=pltpu.CompilerParams(use_tc_tiling_on_sc=False),
    )
    def _sc_gather_kernel(weight_hbm, idx_hbm, out_hbm, idx_v, rows_v, sem):
        wid = lax.axis_index("s") * _NC + lax.axis_index("c")
        base = wid * _BPW
        for j in range(_BPW // _CHUNK):
            off = base + j * _CHUNK
            pltpu.sync_copy(idx_hbm.at[pl.ds(off, _CHUNK)], idx_v)
            pltpu.async_copy(weight_hbm.at[idx_v], rows_v, sem).wait()
            pltpu.sync_copy(rows_v, out_hbm.at[pl.ds(off, _CHUNK)])

    return _sc_gather_kernel


def _sc_gather(weight, idx):
    return _make_sc_gather()(weight, idx)


def _main_call(flat, x2, w2, wt, ids):
    return pl.pallas_call(
        _main_body,
        grid=(N // TB,),
        in_specs=[
            pl.BlockSpec((TB, D), lambda i: (i, 0)),
            pl.BlockSpec((TB, 1), lambda i: (i, 0)),
            pl.BlockSpec((1, K), lambda i: (0, 0)),
            pl.BlockSpec((D, K), lambda i: (0, 0)),
            pl.BlockSpec((1, K), lambda i: (0, 0)),
        ],
        out_specs=[
            pl.BlockSpec((TB, 1), lambda i: (i, 0)),
            pl.BlockSpec((TB, K), lambda i: (i, 0)),
            pl.BlockSpec((1, K), lambda i: (0, 0)),
        ],
        out_shape=[
            jax.ShapeDtypeStruct((N, 1), jnp.int32),
            jax.ShapeDtypeStruct((N, K), jnp.float32),
            jax.ShapeDtypeStruct((1, K), jnp.float32),
        ],
        scratch_shapes=[pltpu.VMEM((1, K), jnp.float32)],
        compiler_params=pltpu.CompilerParams(
            dimension_semantics=("arbitrary",)),
    )(flat, x2, w2, wt, ids)


def _fin_call(flat, quant, cnt):
    return pl.pallas_call(
        _fin_body,
        grid=(N // FB,),
        in_specs=[
            pl.BlockSpec((FB, D), lambda i: (i, 0)),
            pl.BlockSpec((FB, D), lambda i: (i, 0)),
            pl.BlockSpec((1, K), lambda i: (0, 0)),
        ],
        out_specs=[
            pl.BlockSpec((FB, D), lambda i: (i, 0)),
            pl.BlockSpec((1, 1), lambda i: (0, 0)),
            pl.BlockSpec((1, 1), lambda i: (0, 0)),
        ],
        out_shape=[
            jax.ShapeDtypeStruct((N, D), jnp.float32),
            jax.ShapeDtypeStruct((1, 1), jnp.float32),
            jax.ShapeDtypeStruct((1, 1), jnp.float32),
        ],
        scratch_shapes=[pltpu.VMEM((1, 1), jnp.float32)],
        compiler_params=pltpu.CompilerParams(
            dimension_semantics=("arbitrary",)),
    )(flat, quant, cnt)


def kernel(inputs, weight):
    flat = inputs.reshape(N, D)
    x2 = jnp.sum(flat ** 2, axis=1, keepdims=True)
    w2 = jnp.sum(weight ** 2, axis=1)[None, :]
    wt = weight.T
    ids = lax.iota(jnp.float32, K)[None, :]

    idx2d, enc, cnt = _main_call(flat, x2, w2, wt, ids)
    quant = _sc_gather(weight, idx2d.reshape(N))
    qst, loss, perp = _fin_call(flat, quant, cnt)
    return (loss[0, 0], qst.reshape(inputs.shape), perp[0, 0], enc)


# speedup vs baseline: 1.0470x; 1.0470x over previous
import functools

import jax
import jax.numpy as jnp
from jax import lax
from jax.experimental import pallas as pl
from jax.experimental.pallas import tpu as pltpu
from jax.experimental.pallas import tpu_sc as plsc

N = 16384
K = 8192
D = 64
TB = 256

_NC, _NS = 2, 16
_NW = _NC * _NS
_BPW = N // _NW
_CHUNK = 128


def _main_body(x_ref, w2_ref, w_ref, ids_ref,
               idx_ref, enc_ref, cnt_ref, acc_ref):
    i = pl.program_id(0)
    x = x_ref[...]
    mm = lax.dot_general(x, w_ref[...], (((1,), (1,)), ((), ())),
                         preferred_element_type=jnp.float32)
    x2 = jnp.sum(x * x, axis=1, keepdims=True)
    d = (x2 + w2_ref[...]) - 2.0 * mm
    m = jnp.min(d, axis=1, keepdims=True)
    ids = ids_ref[...]
    idf = jnp.min(jnp.where(d == m, ids, jnp.float32(K)),
                  axis=1, keepdims=True)
    oh = jnp.where(ids == idf, jnp.float32(1.0), jnp.float32(0.0))
    enc_ref[...] = oh
    idx_ref[...] = idf.astype(jnp.int32)
    part = jnp.sum(oh, axis=0, keepdims=True)

    @pl.when(i == 0)
    def _():
        acc_ref[...] = part

    @pl.when(i > 0)
    def _():
        acc_ref[...] = acc_ref[...] + part

    @pl.when(i == N // TB - 1)
    def _():
        cnt_ref[...] = acc_ref[...]


FB = 2048


def _fin_body(x_ref, q_ref, cnt_ref, qst_ref, loss_ref, perp_ref, acc_ref):
    i = pl.program_id(0)
    x = x_ref[...]
    q = q_ref[...]
    qst_ref[...] = x + (q - x)
    dlt = q - x
    e = jnp.sum(jnp.sum(dlt * dlt, axis=1, keepdims=True), axis=0,
                keepdims=True)

    @pl.when(i == 0)
    def _():
        acc_ref[...] = e

    @pl.when(i > 0)
    def _():
        acc_ref[...] = acc_ref[...] + e

    @pl.when(i == N // FB - 1)
    def _():
        et = acc_ref[...] * (1.0 / (N * D))
        loss_ref[...] = et + 0.25 * et
        p = cnt_ref[...] * (1.0 / N)
        eps = jnp.float32(jnp.finfo(jnp.float32).eps)
        ent = jnp.sum(p * jnp.log(p + eps), axis=1, keepdims=True)
        perp_ref[...] = jnp.exp(-ent)


@functools.cache
def _make_sc_gather():
    mesh = plsc.VectorSubcoreMesh(
        core_axis_name="c", subcore_axis_name="s",
        num_cores=_NC, num_subcores=_NS)

    @functools.partial(
        pl.kernel,
        mesh=mesh,
        out_type=jax.ShapeDtypeStruct((N, D), jnp.float32),
        scratch_types=[
            pltpu.VMEM((_CHUNK,), jnp.int32),
            pltpu.VMEM((_CHUNK, D), jnp.float32),
            pltpu.SemaphoreType.DMA,
        ],
        compiler_params=pltpu.CompilerParams(use_tc_tiling_on_sc=False),
    )
    def _sc_gather_kernel(weight_hbm, idx_hbm, out_hbm, idx_v, rows_v, sem):
        wid = lax.axis_index("s") * _NC + lax.axis_index("c")
        base = wid * _BPW
        for j in range(_BPW // _CHUNK):
            off = base + j * _CHUNK
            pltpu.sync_copy(idx_hbm.at[pl.ds(off, _CHUNK)], idx_v)
            pltpu.async_copy(weight_hbm.at[idx_v], rows_v, sem).wait()
            pltpu.sync_copy(rows_v, out_hbm.at[pl.ds(off, _CHUNK)])

    return _sc_gather_kernel


def _sc_gather(weight, idx):
    return _make_sc_gather()(weight, idx)


def _main_call(flat, w2, w, ids):
    return pl.pallas_call(
        _main_body,
        grid=(N // TB,),
        in_specs=[
            pl.BlockSpec((TB, D), lambda i: (i, 0)),
            pl.BlockSpec((1, K), lambda i: (0, 0)),
            pl.BlockSpec((K, D), lambda i: (0, 0)),
            pl.BlockSpec((1, K), lambda i: (0, 0)),
        ],
        out_specs=[
            pl.BlockSpec((TB, 1), lambda i: (i, 0)),
            pl.BlockSpec((TB, K), lambda i: (i, 0)),
            pl.BlockSpec((1, K), lambda i: (0, 0)),
        ],
        out_shape=[
            jax.ShapeDtypeStruct((N, 1), jnp.int32),
            jax.ShapeDtypeStruct((N, K), jnp.float32),
            jax.ShapeDtypeStruct((1, K), jnp.float32),
        ],
        scratch_shapes=[pltpu.VMEM((1, K), jnp.float32)],
        compiler_params=pltpu.CompilerParams(
            dimension_semantics=("arbitrary",)),
    )(flat, w2, w, ids)


def _fin_call(flat, quant, cnt):
    return pl.pallas_call(
        _fin_body,
        grid=(N // FB,),
        in_specs=[
            pl.BlockSpec((FB, D), lambda i: (i, 0)),
            pl.BlockSpec((FB, D), lambda i: (i, 0)),
            pl.BlockSpec((1, K), lambda i: (0, 0)),
        ],
        out_specs=[
            pl.BlockSpec((FB, D), lambda i: (i, 0)),
            pl.BlockSpec((1, 1), lambda i: (0, 0)),
            pl.BlockSpec((1, 1), lambda i: (0, 0)),
        ],
        out_shape=[
            jax.ShapeDtypeStruct((N, D), jnp.float32),
            jax.ShapeDtypeStruct((1, 1), jnp.float32),
            jax.ShapeDtypeStruct((1, 1), jnp.float32),
        ],
        scratch_shapes=[pltpu.VMEM((1, 1), jnp.float32)],
        compiler_params=pltpu.CompilerParams(
            dimension_semantics=("arbitrary",)),
    )(flat, quant, cnt)


def kernel(inputs, weight):
    flat = inputs.reshape(N, D)
    w2 = jnp.sum(weight ** 2, axis=1)[None, :]
    ids = lax.iota(jnp.float32, K)[None, :]

    idx2d, enc, cnt = _main_call(flat, w2, weight, ids)
    quant = _sc_gather(weight, idx2d.reshape(N))
    qst, loss, perp = _fin_call(flat, quant, cnt)
    return (loss[0, 0], qst.reshape(inputs.shape), perp[0, 0], enc)


# speedup vs baseline: 1.1530x; 1.1012x over previous
import functools

import jax
import jax.numpy as jnp
from jax import lax
from jax.experimental import pallas as pl
from jax.experimental.pallas import tpu as pltpu
from jax.experimental.pallas import tpu_sc as plsc

N = 16384
K = 8192
D = 64
TB = 256

_NC, _NS = 2, 16
_NW = _NC * _NS
_BPW = N // _NW
_CHUNK = 128


def _main_body(x_ref, w2_ref, w_ref, ids_ref, idx_ref, enc_ref):
    x = x_ref[...]
    mm = lax.dot_general(x, w_ref[...], (((1,), (1,)), ((), ())),
                         preferred_element_type=jnp.float32)
    x2 = jnp.sum(x * x, axis=1, keepdims=True)
    d = (x2 + w2_ref[...]) - 2.0 * mm
    m = jnp.min(d, axis=1, keepdims=True)
    ids = ids_ref[...]
    idf = jnp.min(jnp.where(d == m, ids, jnp.float32(K)),
                  axis=1, keepdims=True)
    oh = jnp.where(ids == idf, jnp.float32(1.0), jnp.float32(0.0))
    enc_ref[...] = oh
    idx_ref[...] = idf.astype(jnp.int32)


FB = 2048


def _fin_body(x_ref, q_ref, cnt2_ref, qst_ref, loss_ref, perp_ref, acc_ref):
    i = pl.program_id(0)
    x = x_ref[...]
    q = q_ref[...]
    qst_ref[...] = x + (q - x)
    dlt = q - x
    e = jnp.sum(jnp.sum(dlt * dlt, axis=1, keepdims=True), axis=0,
                keepdims=True)

    @pl.when(i == 0)
    def _():
        acc_ref[...] = e

    @pl.when(i > 0)
    def _():
        acc_ref[...] = acc_ref[...] + e

    @pl.when(i == N // FB - 1)
    def _():
        et = acc_ref[...] * (1.0 / (N * D))
        csc = cnt2_ref[0:1, :] + cnt2_ref[1:2, :]
        loss_ref[...] = et + 0.25 * et
        p = csc * (1.0 / N)
        eps = jnp.float32(jnp.finfo(jnp.float32).eps)
        ent = jnp.sum(p * jnp.log(p + eps), axis=1, keepdims=True)
        perp_ref[...] = jnp.exp(-ent)


@functools.cache
def _make_sc_gather():
    mesh = plsc.VectorSubcoreMesh(
        core_axis_name="c", subcore_axis_name="s",
        num_cores=_NC, num_subcores=_NS)

    @functools.partial(
        pl.kernel,
        mesh=mesh,
        out_type=[
            jax.ShapeDtypeStruct((N, D), jnp.float32),
            jax.ShapeDtypeStruct((_NC, K), jnp.float32),
        ],
        scratch_types=[
            pltpu.VMEM((_CHUNK,), jnp.int32),
            pltpu.VMEM((_CHUNK, D), jnp.float32),
            pltpu.SemaphoreType.DMA,
            pltpu.VMEM((K,), jnp.float32),
            pltpu.VMEM_SHARED((_NS, K), jnp.float32),
            pltpu.VMEM((K // _NS,), jnp.float32),
            pltpu.VMEM((K // _NS,), jnp.float32),
        ],
        compiler_params=pltpu.CompilerParams(
            use_tc_tiling_on_sc=False, needs_layout_passes=False),
    )
    def _sc_gather_kernel(weight_hbm, idx_hbm, quant_hbm, cnt2_hbm,
                          idx_v, rows_v, sem, bins, shared, tmp, acc):
        cid = lax.axis_index("c")
        sid = lax.axis_index("s")
        wid = sid * _NC + cid
        base = wid * _BPW

        def _zero(j, carry):
            bins[pl.ds(j * 16, 16)] = jnp.zeros((16,), jnp.float32)
            return carry

        lax.fori_loop(0, K // 16, _zero, 0)

        for j in range(_BPW // _CHUNK):
            off = base + j * _CHUNK
            pltpu.sync_copy(idx_hbm.at[pl.ds(off, _CHUNK)], idx_v)
            pltpu.async_copy(weight_hbm.at[idx_v], rows_v, sem).wait()
            pltpu.sync_copy(rows_v, quant_hbm.at[pl.ds(off, _CHUNK)])
            for c in range(_CHUNK // 16):
                iv = idx_v[pl.ds(c * 16, 16)]
                plsc.addupdate_scatter(bins, [iv],
                                       jnp.ones((16,), jnp.float32))

        plsc.subcore_barrier()
        pltpu.sync_copy(bins, shared.at[sid])
        plsc.subcore_barrier()

        w = K // _NS
        col = sid * w
        pltpu.sync_copy(shared.at[0, pl.ds(col, w)], acc)

        def _red(r, carry):
            pltpu.sync_copy(shared.at[r, pl.ds(col, w)], tmp)

            def _add(k2, c2):
                acc[pl.ds(k2 * 16, 16)] = (acc[pl.ds(k2 * 16, 16)]
                                           + tmp[pl.ds(k2 * 16, 16)])
                return c2

            lax.fori_loop(0, w // 16, _add, 0)
            return carry

        lax.fori_loop(1, _NS, _red, 0)
        pltpu.sync_copy(acc, cnt2_hbm.at[cid, pl.ds(col, w)])

    return _sc_gather_kernel


def _sc_gather(weight, idx):
    return _make_sc_gather()(weight, idx)


def _main_call(flat, w2, w, ids):
    return pl.pallas_call(
        _main_body,
        grid=(N // TB,),
        in_specs=[
            pl.BlockSpec((TB, D), lambda i: (i, 0)),
            pl.BlockSpec((1, K), lambda i: (0, 0)),
            pl.BlockSpec((K, D), lambda i: (0, 0)),
            pl.BlockSpec((1, K), lambda i: (0, 0)),
        ],
        out_specs=[
            pl.BlockSpec((TB, 1), lambda i: (i, 0)),
            pl.BlockSpec((TB, K), lambda i: (i, 0)),
        ],
        out_shape=[
            jax.ShapeDtypeStruct((N, 1), jnp.int32),
            jax.ShapeDtypeStruct((N, K), jnp.float32),
        ],
        compiler_params=pltpu.CompilerParams(
            dimension_semantics=("arbitrary",)),
    )(flat, w2, w, ids)


def _fin_call(flat, quant, cnt2):
    return pl.pallas_call(
        _fin_body,
        grid=(N // FB,),
        in_specs=[
            pl.BlockSpec((FB, D), lambda i: (i, 0)),
            pl.BlockSpec((FB, D), lambda i: (i, 0)),
            pl.BlockSpec((_NC, K), lambda i: (0, 0)),
        ],
        out_specs=[
            pl.BlockSpec((FB, D), lambda i: (i, 0)),
            pl.BlockSpec((1, 1), lambda i: (0, 0)),
            pl.BlockSpec((1, 1), lambda i: (0, 0)),
        ],
        out_shape=[
            jax.ShapeDtypeStruct((N, D), jnp.float32),
            jax.ShapeDtypeStruct((1, 1), jnp.float32),
            jax.ShapeDtypeStruct((1, 1), jnp.float32),
        ],
        scratch_shapes=[pltpu.VMEM((1, 1), jnp.float32)],
        compiler_params=pltpu.CompilerParams(
            dimension_semantics=("arbitrary",)),
    )(flat, quant, cnt2)


def kernel(inputs, weight):
    flat = inputs.reshape(N, D)
    w2 = jnp.sum(weight ** 2, axis=1)[None, :]
    ids = lax.iota(jnp.float32, K)[None, :]

    idx2d, enc = _main_call(flat, w2, weight, ids)
    quant, cnt2 = _sc_gather(weight, idx2d.reshape(N))
    qst, loss, perp = _fin_call(flat, quant, cnt2)
    return (loss[0, 0], qst.reshape(inputs.shape), perp[0, 0], enc)
